# baseline (speedup 1.0000x reference)
import jax
import jax.numpy as jnp
from jax.experimental import pallas as pl
from jax.experimental.pallas import tpu as pltpu

NEG_POS_RATIO = 3
_CHUNK = 2048
_SUM_DIMS = (((1,), (0,)), ((), ()))


def _csum(v):
    ones = jnp.ones((1, v.shape[0]), dtype=jnp.float32)
    return jax.lax.dot_general(ones, v, _SUM_DIMS,
                               preferred_element_type=jnp.float32)


def _row_kernel(conf_ref, lab_ref, pred_ref, gt_ref, out_ref, nbg_ref, g_ref):
    b = pl.program_id(0)
    nb = pl.num_programs(0)

    @pl.when(b == 0)
    def _init():
        out_ref[0] = 0.0
        out_ref[1] = 0.0
        out_ref[2] = 0.0
        g_ref[:, :] = jnp.zeros_like(g_ref)

    P = conf_ref.shape[2]
    widths = {}
    for c0 in range(0, P, _CHUNK):
        cw = min(_CHUNK, P - c0)
        if cw not in widths:
            z = jnp.zeros((1, cw), jnp.float32)
            widths[cw] = [z, z, z, z]
        acc = widths[cw]
        sl = pl.ds(c0, cw)
        x = conf_ref[0, :, sl]
        lab = lab_ref[0, :, sl]
        pos = lab > 0
        posf = pos.astype(jnp.float32)

        m = jnp.max(x, axis=0, keepdims=True)
        e = jnp.exp(x - m)
        lse = m + jnp.log(_csum(e))

        x0 = x[0:1, :]
        cls_iota = jax.lax.broadcasted_iota(jnp.int32, x.shape, 0)
        xl = _csum(jnp.where(cls_iota == lab, x, 0.0))

        bg = lse - x0
        acc[0] += bg + (x0 - xl) * posf
        acc[1] += (lse - xl) * posf
        acc[2] += posf
        nbg_ref[0:1, sl] = jnp.where(pos, -jnp.inf, bg)

        d = pred_ref[0, :, sl] - gt_ref[0, :, sl]
        ad = jnp.abs(d)
        sl1 = jnp.where(ad < 1.0, 0.5 * d * d, ad - 0.5)
        acc[3] += _csum(sl1) * posf

    npos = 0.0
    for acc in widths.values():
        npos += jnp.sum(acc[2])
    nneg = P - npos
    k = NEG_POS_RATIO * npos

    @pl.when(k >= nneg)
    def _fast():
        base = 0
        for cw, acc in widths.items():
            g_ref[1:2, pl.ds(base, cw)] += acc[0]
            base += cw

    @pl.when(k < nneg)
    def _slow():
        ce_pos = 0.0
        for acc in widths.values():
            ce_pos += jnp.sum(acc[1])
        negbg = nbg_ref[0:1, :]
        finite = jnp.where(negbg == -jnp.inf, jnp.inf, negbg)
        lo0 = jnp.min(finite) - 1.0
        hi0 = jnp.max(negbg)

        def _bisect(_, carry):
            lo, hi = carry
            mid = 0.5 * (lo + hi)
            c = jnp.sum((negbg > mid).astype(jnp.float32))
            return jnp.where(c > k, mid, lo), jnp.where(c > k, hi, mid)

        lo, hi = jax.lax.fori_loop(0, 48, _bisect, (lo0, hi0))
        sel_hi = negbg > hi
        c1 = jnp.sum(sel_hi.astype(jnp.float32))
        s1 = jnp.sum(jnp.where(sel_hi, negbg, 0.0))
        r = k - c1
        band = jnp.logical_and(negbg <= hi, negbg > lo)
        idx = jax.lax.broadcasted_iota(jnp.int32, band.shape, 1)

        def _ibisect(_, carry):
            jlo, jhi = carry
            jm = (jlo + jhi) // 2
            c = jnp.sum(jnp.logical_and(band, idx < jm).astype(jnp.float32))
            return jnp.where(c <= r, jm, jlo), jnp.where(c <= r, jhi, jm)

        jlo, _ = jax.lax.fori_loop(0, 16, _ibisect, (0, P + 1))
        s2 = jnp.sum(jnp.where(jnp.logical_and(band, idx < jlo), negbg, 0.0))
        out_ref[1] += ce_pos + s1 + s2

    base = 0
    for cw, acc in widths.items():
        g_ref[0:1, pl.ds(base, cw)] += acc[3]
        base += cw
    out_ref[2] += npos

    @pl.when(b == nb - 1)
    def _finish():
        out_ref[0] += jnp.sum(g_ref[0:1, :])
        out_ref[1] += jnp.sum(g_ref[1:2, :])


def _slice_sums(confidence, predicted_locations, labels, gt_locations):
    B, P, C = confidence.shape
    conf_t = jnp.swapaxes(confidence, 1, 2)
    pred_t = jnp.swapaxes(predicted_locations, 1, 2)
    gt_t = jnp.swapaxes(gt_locations, 1, 2)
    lab3 = labels.reshape(B, 1, P)
    return pl.pallas_call(
        _row_kernel,
        grid=(B,),
        in_specs=[
            pl.BlockSpec((1, C, P), lambda b: (b, 0, 0)),
            pl.BlockSpec((1, 1, P), lambda b: (b, 0, 0)),
            pl.BlockSpec((1, 4, P), lambda b: (b, 0, 0)),
            pl.BlockSpec((1, 4, P), lambda b: (b, 0, 0)),
        ],
        out_specs=pl.BlockSpec(memory_space=pltpu.SMEM),
        out_shape=jax.ShapeDtypeStruct((3,), jnp.float32),
        scratch_shapes=[
            pltpu.VMEM((8, P), jnp.float32),
            pltpu.VMEM((8, P), jnp.float32),
        ],
    )(conf_t, lab3, pred_t, gt_t)


_NSPLIT = 2


@jax.jit
def kernel(confidence, predicted_locations, labels, gt_locations):
    B = confidence.shape[0]
    step = B // _NSPLIT
    sums = 0.0
    for i in range(_NSPLIT):
        s = slice(i * step, (i + 1) * step)
        sums = sums + _slice_sums(confidence[s], predicted_locations[s],
                                  labels[s], gt_locations[s])
    num_pos = sums[2]
    return sums[0] / num_pos, sums[1] / num_pos


# speedup vs baseline: 1.3924x; 1.3924x over previous
import jax
import jax.numpy as jnp
from jax.experimental import pallas as pl
from jax.experimental.pallas import tpu as pltpu

NEG_POS_RATIO = 3
_CHUNK = 2048
_SUM_DIMS = (((1,), (0,)), ((), ()))


def _csum(v):
    ones = jnp.ones((1, v.shape[0]), dtype=jnp.float32)
    return jax.lax.dot_general(ones, v, _SUM_DIMS,
                               preferred_element_type=jnp.float32)


def _row_kernel(conf_ref, lab_ref, pred_ref, gt_ref, out_ref, nbg_ref, g_ref):
    b = pl.program_id(0)
    nb = pl.num_programs(0)

    @pl.when(b == 0)
    def _init():
        out_ref[0] = 0.0
        out_ref[1] = 0.0
        out_ref[2] = 0.0
        g_ref[:, :] = jnp.zeros_like(g_ref)

    P = conf_ref.shape[2]
    widths = {}
    for c0 in range(0, P, _CHUNK):
        cw = min(_CHUNK, P - c0)
        if cw not in widths:
            z = jnp.zeros((1, cw), jnp.float32)
            widths[cw] = [z, z, z, z]
        acc = widths[cw]
        sl = pl.ds(c0, cw)
        x = conf_ref[0, :, sl]
        lab = lab_ref[0, :, sl]
        pos = lab > 0
        posf = pos.astype(jnp.float32)

        m = jnp.max(x, axis=0, keepdims=True)
        e = jnp.exp(x - m)
        lse = m + jnp.log(_csum(e))

        x0 = x[0:1, :]
        cls_iota = jax.lax.broadcasted_iota(jnp.int32, x.shape, 0)
        xl = _csum(jnp.where(cls_iota == lab, x, 0.0))

        bg = lse - x0
        acc[0] += bg + (x0 - xl) * posf
        acc[1] += (lse - xl) * posf
        acc[2] += posf
        nbg_ref[0:1, sl] = jnp.where(pos, -jnp.inf, bg)

        d = pred_ref[0, :, sl] - gt_ref[0, :, sl]
        ad = jnp.abs(d)
        sl1 = jnp.where(ad < 1.0, 0.5 * d * d, ad - 0.5)
        acc[3] += _csum(sl1) * posf

    npos = 0.0
    for acc in widths.values():
        npos += jnp.sum(acc[2])
    nneg = P - npos
    k = NEG_POS_RATIO * npos

    @pl.when(k >= nneg)
    def _fast():
        base = 0
        for cw, acc in widths.items():
            g_ref[1:2, pl.ds(base, cw)] += acc[0]
            base += cw

    @pl.when(k < nneg)
    def _slow():
        ce_pos = 0.0
        for acc in widths.values():
            ce_pos += jnp.sum(acc[1])
        negbg = nbg_ref[0:1, :]
        finite = jnp.where(negbg == -jnp.inf, jnp.inf, negbg)
        lo0 = jnp.min(finite) - 1.0
        hi0 = jnp.max(negbg)

        def _bisect(_, carry):
            lo, hi = carry
            mid = 0.5 * (lo + hi)
            c = jnp.sum((negbg > mid).astype(jnp.float32))
            return jnp.where(c > k, mid, lo), jnp.where(c > k, hi, mid)

        lo, hi = jax.lax.fori_loop(0, 48, _bisect, (lo0, hi0))
        sel_hi = negbg > hi
        c1 = jnp.sum(sel_hi.astype(jnp.float32))
        s1 = jnp.sum(jnp.where(sel_hi, negbg, 0.0))
        r = k - c1
        band = jnp.logical_and(negbg <= hi, negbg > lo)
        idx = jax.lax.broadcasted_iota(jnp.int32, band.shape, 1)

        def _ibisect(_, carry):
            jlo, jhi = carry
            jm = (jlo + jhi) // 2
            c = jnp.sum(jnp.logical_and(band, idx < jm).astype(jnp.float32))
            return jnp.where(c <= r, jm, jlo), jnp.where(c <= r, jhi, jm)

        jlo, _ = jax.lax.fori_loop(0, 16, _ibisect, (0, P + 1))
        s2 = jnp.sum(jnp.where(jnp.logical_and(band, idx < jlo), negbg, 0.0))
        out_ref[1] += ce_pos + s1 + s2

    base = 0
    for cw, acc in widths.items():
        g_ref[0:1, pl.ds(base, cw)] += acc[3]
        base += cw
    out_ref[2] += npos

    @pl.when(b == nb - 1)
    def _finish():
        out_ref[0] += jnp.sum(g_ref[0:1, :])
        out_ref[1] += jnp.sum(g_ref[1:2, :])


def _slice_sums(confidence, predicted_locations, labels, gt_locations):
    B, P, C = confidence.shape
    conf_t = jnp.swapaxes(confidence, 1, 2)
    pred_t = jnp.swapaxes(predicted_locations, 1, 2)
    gt_t = jnp.swapaxes(gt_locations, 1, 2)
    lab3 = labels.reshape(B, 1, P)
    return pl.pallas_call(
        _row_kernel,
        grid=(B,),
        in_specs=[
            pl.BlockSpec((1, C, P), lambda b: (b, 0, 0)),
            pl.BlockSpec((1, 1, P), lambda b: (b, 0, 0)),
            pl.BlockSpec((1, 4, P), lambda b: (b, 0, 0)),
            pl.BlockSpec((1, 4, P), lambda b: (b, 0, 0)),
        ],
        out_specs=pl.BlockSpec(memory_space=pltpu.SMEM),
        out_shape=jax.ShapeDtypeStruct((3,), jnp.float32),
        scratch_shapes=[
            pltpu.VMEM((8, P), jnp.float32),
            pltpu.VMEM((8, P), jnp.float32),
        ],
    )(conf_t, lab3, pred_t, gt_t)


_NSPLIT = 1


@jax.jit
def kernel(confidence, predicted_locations, labels, gt_locations):
    B = confidence.shape[0]
    step = B // _NSPLIT
    sums = 0.0
    for i in range(_NSPLIT):
        s = slice(i * step, (i + 1) * step)
        sums = sums + _slice_sums(confidence[s], predicted_locations[s],
                                  labels[s], gt_locations[s])
    num_pos = sums[2]
    return sums[0] / num_pos, sums[1] / num_pos
